# baseline (speedup 1.0000x reference)
import functools

import jax
import jax.numpy as jnp
from jax import lax
from jax.experimental import pallas as pl
from jax.experimental.pallas import tpu as pltpu
from jax.experimental.pallas import tpu_sc as plsc

N = 10000
E = 320000
D = 128

NC = 2
NS = 16
NW = NC * NS
EPW = E // NW
PADL = 240
EPWP = EPW + PADL
CH = 128
NCH = EPWP // CH
NA = N + 8
RMAIN = 640
RLAST = N - (NS - 1) * RMAIN
ZROWS = 80

_MESH = plsc.VectorSubcoreMesh(
    core_axis_name="c", subcore_axis_name="s", num_cores=NC, num_subcores=NS
)


def _fill2d(ref, rows, width, value):
    vec = jnp.full((16,), value, dtype=jnp.float32)

    def row(i, _):
        for cstart in range(0, width, 16):
            ref[i, pl.ds(cstart, 16)] = vec
        return 0

    lax.fori_loop(0, rows, row, 0)


def _zero_acc_rows(zsrc, acc, s):
    start = pl.multiple_of(RMAIN * s, RMAIN)

    @pl.when(s < NS - 1)
    def _():
        for k in range(RMAIN // ZROWS):
            off = pl.multiple_of(start + k * ZROWS, ZROWS)
            pltpu.sync_copy(zsrc, acc.at[pl.ds(off, ZROWS)])

    @pl.when(s == NS - 1)
    def _():
        for k in range(RLAST // ZROWS):
            off = pl.multiple_of(start + k * ZROWS, ZROWS)
            pltpu.sync_copy(zsrc, acc.at[pl.ds(off, ZROWS)])


def _copy_out_rows(acc, out2d, s):
    start = pl.multiple_of(RMAIN * s, RMAIN)

    @pl.when(s < NS - 1)
    def _():
        pltpu.sync_copy(acc.at[pl.ds(start, RMAIN)], out2d.at[pl.ds(start, RMAIN)])

    @pl.when(s == NS - 1)
    def _():
        pltpu.sync_copy(acc.at[pl.ds(start, RLAST)], out2d.at[pl.ds(start, RLAST)])


def _deg_body(e_hbm, out_hbm, didx, ones_v, sem, acc):
    c = lax.axis_index("c")
    s = lax.axis_index("s")
    w = c * NS + s

    _fill2d(ones_v, CH, 16, 0.0)
    _zero_acc_rows(ones_v.at[pl.ds(0, ZROWS)], acc, s)
    _fill2d(ones_v, CH, 16, 1.0)
    pltpu.sync_copy(e_hbm.at[1, w], didx)
    plsc.subcore_barrier()

    def chunk(j, _):
        pltpu.async_copy(ones_v, acc.at[didx.at[pl.ds(j * CH, CH)]], sem, add=True)
        return 0

    lax.fori_loop(0, NCH, chunk, 0)

    def drain(j, _):
        pltpu.make_async_copy(ones_v, acc.at[didx.at[pl.ds(j * CH, CH)]], sem).wait()
        return 0

    lax.fori_loop(0, NCH, drain, 0)
    plsc.subcore_barrier()
    _copy_out_rows(acc, out_hbm.at[c], s)


_deg = functools.partial(
    pl.kernel,
    out_type=jax.ShapeDtypeStruct((NC, N, 16), jnp.float32),
    mesh=_MESH,
    scratch_types=[
        pltpu.VMEM((EPWP,), jnp.int32),
        pltpu.VMEM((CH, 16), jnp.float32),
        pltpu.SemaphoreType.DMA,
        pltpu.VMEM_SHARED((NA, 16), jnp.float32),
    ],
    compiler_params=pltpu.CompilerParams(use_tc_tiling_on_sc=False),
)(_deg_body)


NH = 2
HW = D // NH


NBUF = 6
AHEAD = NBUF // 2
MAIN = (NCH // NBUF) * NBUF


def _copy_out_cols(acc, out2d, s, off):
    start = pl.multiple_of(RMAIN * s, RMAIN)

    @pl.when(s < NS - 1)
    def _():
        pltpu.sync_copy(
            acc.at[pl.ds(start, RMAIN)],
            out2d.at[pl.ds(start, RMAIN), pl.ds(off, HW)],
        )

    @pl.when(s == NS - 1)
    def _():
        pltpu.sync_copy(
            acc.at[pl.ds(start, RLAST)],
            out2d.at[pl.ds(start, RLAST), pl.ds(off, HW)],
        )


def _scat_body(
    e_hbm, y_hbm, out_hbm,
    sidx, didx, *rest,
):
    bufs = rest[:NBUF]
    gsems = rest[NBUF : 2 * NBUF]
    ssems = rest[2 * NBUF : 3 * NBUF]
    acc = rest[3 * NBUF]
    c = lax.axis_index("c")
    s = lax.axis_index("s")
    w = c * NS + s
    buf0 = bufs[0]

    pltpu.sync_copy(e_hbm.at[0, w], sidx)
    pltpu.sync_copy(e_hbm.at[1, w], didx)

    def dbl(i, _):
        v = sidx[pl.ds(i * 16, 16)]
        sidx[pl.ds(i * 16, 16)] = v + v
        return 0

    lax.fori_loop(0, EPWP // 16, dbl, 0)

    for half in range(NH):
        y_h = y_hbm.at[pl.ds(half, 2 * N - 1)]

        _fill2d(buf0, CH, HW, 0.0)
        _zero_acc_rows(buf0.at[pl.ds(0, ZROWS)], acc, s)

        for j0 in range(AHEAD):
            pltpu.async_copy(y_h.at[sidx.at[pl.ds((j0) * CH, CH)]], bufs[j0 % NBUF], gsems[j0 % NBUF])
        plsc.subcore_barrier()

        def step(jj, _):
            for b in range(NBUF):
                j = jj * NBUF + b
                b2 = (b + AHEAD) % NBUF
                pltpu.make_async_copy(y_h.at[sidx.at[pl.ds((j) * CH, CH)]], bufs[b], gsems[b]).wait()
                pltpu.async_copy(bufs[b], acc.at[didx.at[pl.ds((j) * CH, CH)]], ssems[b], add=True)

                @pl.when(j >= AHEAD)
                def _():
                    pltpu.make_async_copy(
                        bufs[b2], acc.at[didx.at[pl.ds((j - AHEAD) * CH, CH)]], ssems[b2]
                    ).wait()

                @pl.when(j + AHEAD < NCH)
                def _():
                    pltpu.async_copy(y_h.at[sidx.at[pl.ds((j + AHEAD) * CH, CH)]], bufs[b2], gsems[b2])

            return 0

        lax.fori_loop(0, MAIN // NBUF, step, 0)
        for j in range(MAIN, NCH):
            b = j % NBUF
            pltpu.make_async_copy(y_h.at[sidx.at[pl.ds((j) * CH, CH)]], bufs[b], gsems[b]).wait()
            pltpu.async_copy(bufs[b], acc.at[didx.at[pl.ds((j) * CH, CH)]], ssems[b], add=True)
        for j in range(MAIN - AHEAD, NCH):
            b = j % NBUF
            pltpu.make_async_copy(bufs[b], acc.at[didx.at[pl.ds((j) * CH, CH)]], ssems[b]).wait()
        plsc.subcore_barrier()
        _copy_out_cols(acc, out_hbm.at[c], s, half * HW)


_scat = functools.partial(
    pl.kernel,
    out_type=jax.ShapeDtypeStruct((NC, N, D), jnp.float32),
    mesh=_MESH,
    scratch_types=[
        pltpu.VMEM((EPWP,), jnp.int32),
        pltpu.VMEM((EPWP,), jnp.int32),
        *([pltpu.VMEM((CH, HW), jnp.float32)] * NBUF),
        *([pltpu.SemaphoreType.DMA] * (2 * NBUF)),
        pltpu.VMEM_SHARED((NA, HW), jnp.float32),
    ],
    compiler_params=pltpu.CompilerParams(use_tc_tiling_on_sc=False),
)(_scat_body)


R = 1000
GRID = N // R


def _dinv_block(degp_ref):
    d = degp_ref[0, :, 0:1] + degp_ref[1, :, 0:1] + 1.0
    return lax.rsqrt(d)


def _tcmm_body(x_ref, w_ref, xw_ref):
    xw_ref[...] = jnp.dot(
        x_ref[...], w_ref[...], preferred_element_type=jnp.float32
    )


_tcmm = pl.pallas_call(
    _tcmm_body,
    grid=(GRID,),
    in_specs=[
        pl.BlockSpec((R, D), lambda i: (i, 0)),
        pl.BlockSpec((D, D), lambda i: (0, 0)),
    ],
    out_specs=pl.BlockSpec((R, D), lambda i: (i, 0)),
    out_shape=jax.ShapeDtypeStruct((N, D), jnp.float32),
)


def _tc1_body(xw_ref, degp_ref, y_ref):
    dinv = _dinv_block(degp_ref)
    y_ref[...] = xw_ref[...] * dinv


_tc1 = pl.pallas_call(
    _tc1_body,
    grid=(GRID,),
    in_specs=[
        pl.BlockSpec((R, D), lambda i: (i, 0)),
        pl.BlockSpec((NC, R, 16), lambda i: (0, i, 0)),
    ],
    out_specs=pl.BlockSpec((R, D), lambda i: (i, 0)),
    out_shape=jax.ShapeDtypeStruct((N, D), jnp.float32),
)


def _tc2_body(y1_ref, acc_ref, degp_ref, b_ref, w_ref, y2_ref):
    dinv = _dinv_block(degp_ref)
    tot = acc_ref[0] + acc_ref[1] + y1_ref[...]
    h = jnp.maximum(tot * dinv + b_ref[...], 0.0)
    y2_ref[...] = (
        jnp.dot(h, w_ref[...], preferred_element_type=jnp.float32) * dinv
    )


_tc2 = pl.pallas_call(
    _tc2_body,
    grid=(GRID,),
    in_specs=[
        pl.BlockSpec((R, D), lambda i: (i, 0)),
        pl.BlockSpec((NC, R, D), lambda i: (0, i, 0)),
        pl.BlockSpec((NC, R, 16), lambda i: (0, i, 0)),
        pl.BlockSpec((1, D), lambda i: (0, 0)),
        pl.BlockSpec((D, D), lambda i: (0, 0)),
    ],
    out_specs=pl.BlockSpec((R, D), lambda i: (i, 0)),
    out_shape=jax.ShapeDtypeStruct((N, D), jnp.float32),
)


def _tc3_body(y2_ref, acc_ref, degp_ref, b_ref, o_ref):
    dinv = _dinv_block(degp_ref)
    tot = acc_ref[0] + acc_ref[1] + y2_ref[...]
    t = tot * dinv + b_ref[...]
    o_ref[...] = jax.nn.sigmoid(t) * 0.8 + 0.1


_tc3 = pl.pallas_call(
    _tc3_body,
    grid=(GRID,),
    in_specs=[
        pl.BlockSpec((R, D), lambda i: (i, 0)),
        pl.BlockSpec((NC, R, D), lambda i: (0, i, 0)),
        pl.BlockSpec((NC, R, 16), lambda i: (0, i, 0)),
        pl.BlockSpec((1, D), lambda i: (0, 0)),
    ],
    out_specs=pl.BlockSpec((R, D), lambda i: (i, 0)),
    out_shape=jax.ShapeDtypeStruct((N, D), jnp.float32),
)


@jax.jit
def kernel(x, edge_index, W1, b1, W2, b2):
    e3 = edge_index.reshape(2, NW, EPW)
    pad = jnp.concatenate(
        [
            jnp.zeros((1, NW, PADL), jnp.int32),
            jnp.full((1, NW, PADL), N, jnp.int32),
        ],
        axis=0,
    )
    edge3 = jnp.concatenate([e3, pad], axis=2)
    b1r = b1.reshape(1, D)
    b2r = b2.reshape(1, D)

    xw1 = _tcmm(x, W1)
    degp = _deg(edge3)
    y1 = _tc1(xw1, degp)
    acc1 = _scat(edge3, y1.reshape(2 * N, HW))
    y2 = _tc2(y1, acc1, degp, b1r, W2)
    acc2 = _scat(edge3, y2.reshape(2 * N, HW))
    return _tc3(y2, acc2, degp, b2r)


# speedup vs baseline: 2.8071x; 2.8071x over previous
import functools

import jax
import jax.numpy as jnp
from jax import lax
from jax.experimental import pallas as pl
from jax.experimental.pallas import tpu as pltpu
from jax.experimental.pallas import tpu_sc as plsc

N = 10000
E = 320000
D = 128

NC = 2
NS = 16
NW = NC * NS
EPW = E // NW
CH = 125
NCH = EPW // CH
RMAIN = 640
RLAST = N - (NS - 1) * RMAIN
ZROWS = 80

_MESH = plsc.VectorSubcoreMesh(
    core_axis_name="c", subcore_axis_name="s", num_cores=NC, num_subcores=NS
)


def _fill2d(ref, rows, width, value):
    vec = jnp.full((16,), value, dtype=jnp.float32)

    def row(i, _):
        for cstart in range(0, width, 16):
            ref[i, pl.ds(cstart, 16)] = vec
        return 0

    lax.fori_loop(0, rows, row, 0)


def _zero_acc_rows(zsrc, acc, s):
    start = pl.multiple_of(RMAIN * s, RMAIN)

    @pl.when(s < NS - 1)
    def _():
        for k in range(RMAIN // ZROWS):
            off = pl.multiple_of(start + k * ZROWS, ZROWS)
            pltpu.sync_copy(zsrc, acc.at[pl.ds(off, ZROWS)])

    @pl.when(s == NS - 1)
    def _():
        for k in range(RLAST // ZROWS):
            off = pl.multiple_of(start + k * ZROWS, ZROWS)
            pltpu.sync_copy(zsrc, acc.at[pl.ds(off, ZROWS)])


def _copy_out_rows(acc, out2d, s):
    start = pl.multiple_of(RMAIN * s, RMAIN)

    @pl.when(s < NS - 1)
    def _():
        pltpu.sync_copy(acc.at[pl.ds(start, RMAIN)], out2d.at[pl.ds(start, RMAIN)])

    @pl.when(s == NS - 1)
    def _():
        pltpu.sync_copy(acc.at[pl.ds(start, RLAST)], out2d.at[pl.ds(start, RLAST)])


def _deg_body(e_hbm, out_hbm, didx, ones_v, sem, acc):
    c = lax.axis_index("c")
    s = lax.axis_index("s")
    w = c * NS + s

    _fill2d(ones_v, CH, 16, 0.0)
    _zero_acc_rows(ones_v.at[pl.ds(0, ZROWS)], acc, s)
    _fill2d(ones_v, CH, 16, 1.0)
    pltpu.sync_copy(e_hbm.at[1, w], didx)
    plsc.subcore_barrier()

    def chunk(j, _):
        pltpu.async_copy(ones_v, acc.at[didx.at[j]], sem, add=True)
        return 0

    lax.fori_loop(0, NCH, chunk, 0)

    def drain(j, _):
        pltpu.make_async_copy(ones_v, acc.at[didx.at[j]], sem).wait()
        return 0

    lax.fori_loop(0, NCH, drain, 0)
    plsc.subcore_barrier()
    _copy_out_rows(acc, out_hbm.at[c], s)


_deg = functools.partial(
    pl.kernel,
    out_type=jax.ShapeDtypeStruct((NC, N, 16), jnp.float32),
    mesh=_MESH,
    scratch_types=[
        pltpu.VMEM((NCH, CH), jnp.int32),
        pltpu.VMEM((CH, 16), jnp.float32),
        pltpu.SemaphoreType.DMA,
        pltpu.VMEM_SHARED((N, 16), jnp.float32),
    ],
    compiler_params=pltpu.CompilerParams(use_tc_tiling_on_sc=False),
)(_deg_body)


NH = 2
HW = D // NH


NBUF = 6
AHEAD = NBUF // 2
MAIN = (NCH // NBUF) * NBUF


def _copy_out_cols(acc, out2d, s, off):
    start = pl.multiple_of(RMAIN * s, RMAIN)

    @pl.when(s < NS - 1)
    def _():
        pltpu.sync_copy(
            acc.at[pl.ds(start, RMAIN)],
            out2d.at[pl.ds(start, RMAIN), pl.ds(off, HW)],
        )

    @pl.when(s == NS - 1)
    def _():
        pltpu.sync_copy(
            acc.at[pl.ds(start, RLAST)],
            out2d.at[pl.ds(start, RLAST), pl.ds(off, HW)],
        )


def _scat_body(
    e_hbm, y_hbm, out_hbm,
    sidx, didx, *rest,
):
    bufs = rest[:NBUF]
    gsems = rest[NBUF : 2 * NBUF]
    ssems = rest[2 * NBUF : 3 * NBUF]
    acc = rest[3 * NBUF]
    c = lax.axis_index("c")
    s = lax.axis_index("s")
    w = c * NS + s
    buf0 = bufs[0]

    pltpu.sync_copy(e_hbm.at[0, w], sidx)
    pltpu.sync_copy(e_hbm.at[1, w], didx)

    for half in range(NH):
        y_h = y_hbm.at[pl.ds(half, 2 * N - 1)]

        _fill2d(buf0, CH, HW, 0.0)
        _zero_acc_rows(buf0.at[pl.ds(0, ZROWS)], acc, s)

        for j0 in range(AHEAD):
            pltpu.async_copy(y_h.at[sidx.at[j0]], bufs[j0 % NBUF], gsems[j0 % NBUF])
        plsc.subcore_barrier()

        def step(jj, _):
            for b in range(NBUF):
                j = jj * NBUF + b
                b2 = (b + AHEAD) % NBUF
                pltpu.make_async_copy(y_h.at[sidx.at[j]], bufs[b], gsems[b]).wait()
                pltpu.async_copy(bufs[b], acc.at[didx.at[j]], ssems[b], add=True)

                @pl.when(j >= AHEAD)
                def _():
                    pltpu.make_async_copy(
                        bufs[b2], acc.at[didx.at[j - AHEAD]], ssems[b2]
                    ).wait()

                @pl.when(j + AHEAD < NCH)
                def _():
                    pltpu.async_copy(y_h.at[sidx.at[j + AHEAD]], bufs[b2], gsems[b2])

            return 0

        lax.fori_loop(0, MAIN // NBUF, step, 0)
        for j in range(MAIN, NCH):
            b = j % NBUF
            pltpu.make_async_copy(y_h.at[sidx.at[j]], bufs[b], gsems[b]).wait()
            pltpu.async_copy(bufs[b], acc.at[didx.at[j]], ssems[b], add=True)
        for j in range(MAIN - AHEAD, NCH):
            b = j % NBUF
            pltpu.make_async_copy(bufs[b], acc.at[didx.at[j]], ssems[b]).wait()
        plsc.subcore_barrier()
        _copy_out_cols(acc, out_hbm.at[c], s, half * HW)


_scat = functools.partial(
    pl.kernel,
    out_type=jax.ShapeDtypeStruct((NC, N, D), jnp.float32),
    mesh=_MESH,
    scratch_types=[
        pltpu.VMEM((NCH, CH), jnp.int32),
        pltpu.VMEM((NCH, CH), jnp.int32),
        *([pltpu.VMEM((CH, HW), jnp.float32)] * NBUF),
        *([pltpu.SemaphoreType.DMA] * (2 * NBUF)),
        pltpu.VMEM_SHARED((N, HW), jnp.float32),
    ],
    compiler_params=pltpu.CompilerParams(use_tc_tiling_on_sc=False),
)(_scat_body)


R = 1000
GRID = N // R


def _dinv_block(degp_ref):
    d = degp_ref[0, :, 0:1] + degp_ref[1, :, 0:1] + 1.0
    return lax.rsqrt(d)


def _tcmm_body(x_ref, w_ref, xw_ref):
    xw_ref[...] = jnp.dot(
        x_ref[...], w_ref[...], preferred_element_type=jnp.float32
    )


_tcmm = pl.pallas_call(
    _tcmm_body,
    grid=(GRID,),
    in_specs=[
        pl.BlockSpec((R, D), lambda i: (i, 0)),
        pl.BlockSpec((D, D), lambda i: (0, 0)),
    ],
    out_specs=pl.BlockSpec((R, D), lambda i: (i, 0)),
    out_shape=jax.ShapeDtypeStruct((N, D), jnp.float32),
)


def _tc1_body(xw_ref, degp_ref, y_ref):
    dinv = _dinv_block(degp_ref)
    y_ref[...] = xw_ref[...] * dinv


_tc1 = pl.pallas_call(
    _tc1_body,
    grid=(GRID,),
    in_specs=[
        pl.BlockSpec((R, D), lambda i: (i, 0)),
        pl.BlockSpec((NC, R, 16), lambda i: (0, i, 0)),
    ],
    out_specs=pl.BlockSpec((R, D), lambda i: (i, 0)),
    out_shape=jax.ShapeDtypeStruct((N, D), jnp.float32),
)


def _tc2_body(y1_ref, acc_ref, degp_ref, b_ref, w_ref, y2_ref):
    dinv = _dinv_block(degp_ref)
    tot = acc_ref[0] + acc_ref[1] + y1_ref[...]
    h = jnp.maximum(tot * dinv + b_ref[...], 0.0)
    y2_ref[...] = (
        jnp.dot(h, w_ref[...], preferred_element_type=jnp.float32) * dinv
    )


_tc2 = pl.pallas_call(
    _tc2_body,
    grid=(GRID,),
    in_specs=[
        pl.BlockSpec((R, D), lambda i: (i, 0)),
        pl.BlockSpec((NC, R, D), lambda i: (0, i, 0)),
        pl.BlockSpec((NC, R, 16), lambda i: (0, i, 0)),
        pl.BlockSpec((1, D), lambda i: (0, 0)),
        pl.BlockSpec((D, D), lambda i: (0, 0)),
    ],
    out_specs=pl.BlockSpec((R, D), lambda i: (i, 0)),
    out_shape=jax.ShapeDtypeStruct((N, D), jnp.float32),
)


def _tc3_body(y2_ref, acc_ref, degp_ref, b_ref, o_ref):
    dinv = _dinv_block(degp_ref)
    tot = acc_ref[0] + acc_ref[1] + y2_ref[...]
    t = tot * dinv + b_ref[...]
    o_ref[...] = jax.nn.sigmoid(t) * 0.8 + 0.1


_tc3 = pl.pallas_call(
    _tc3_body,
    grid=(GRID,),
    in_specs=[
        pl.BlockSpec((R, D), lambda i: (i, 0)),
        pl.BlockSpec((NC, R, D), lambda i: (0, i, 0)),
        pl.BlockSpec((NC, R, 16), lambda i: (0, i, 0)),
        pl.BlockSpec((1, D), lambda i: (0, 0)),
    ],
    out_specs=pl.BlockSpec((R, D), lambda i: (i, 0)),
    out_shape=jax.ShapeDtypeStruct((N, D), jnp.float32),
)


@jax.jit
def kernel(x, edge_index, W1, b1, W2, b2):
    flat = jnp.concatenate([edge_index[0] * 2, edge_index[1]])
    edge3 = flat.reshape(2, NW, NCH, CH)
    b1r = b1.reshape(1, D)
    b2r = b2.reshape(1, D)

    xw1 = _tcmm(x, W1)
    degp = _deg(edge3)
    y1 = _tc1(xw1, degp)
    acc1 = _scat(edge3, y1.reshape(2 * N, HW))
    y2 = _tc2(y1, acc1, degp, b1r, W2)
    acc2 = _scat(edge3, y2.reshape(2 * N, HW))
    return _tc3(y2, acc2, degp, b2r)


# speedup vs baseline: 2.8610x; 1.0192x over previous
import functools

import jax
import jax.numpy as jnp
from jax import lax
from jax.experimental import pallas as pl
from jax.experimental.pallas import tpu as pltpu
from jax.experimental.pallas import tpu_sc as plsc

N = 10000
E = 320000
D = 128

NC = 2
NS = 16
NW = NC * NS
EPW = E // NW
CH = 125
NCH = EPW // CH
RMAIN = 640
RLAST = N - (NS - 1) * RMAIN
ZROWS = 80

_MESH = plsc.VectorSubcoreMesh(
    core_axis_name="c", subcore_axis_name="s", num_cores=NC, num_subcores=NS
)


def _fill2d(ref, rows, width, value):
    vec = jnp.full((16,), value, dtype=jnp.float32)

    def row(i, _):
        for cstart in range(0, width, 16):
            ref[i, pl.ds(cstart, 16)] = vec
        return 0

    lax.fori_loop(0, rows, row, 0)


def _zero_acc_rows(zsrc, acc, s):
    start = pl.multiple_of(RMAIN * s, RMAIN)

    @pl.when(s < NS - 1)
    def _():
        for k in range(RMAIN // ZROWS):
            off = pl.multiple_of(start + k * ZROWS, ZROWS)
            pltpu.sync_copy(zsrc, acc.at[pl.ds(off, ZROWS)])

    @pl.when(s == NS - 1)
    def _():
        for k in range(RLAST // ZROWS):
            off = pl.multiple_of(start + k * ZROWS, ZROWS)
            pltpu.sync_copy(zsrc, acc.at[pl.ds(off, ZROWS)])


def _copy_out_rows(acc, out2d, s):
    start = pl.multiple_of(RMAIN * s, RMAIN)

    @pl.when(s < NS - 1)
    def _():
        pltpu.sync_copy(acc.at[pl.ds(start, RMAIN)], out2d.at[pl.ds(start, RMAIN)])

    @pl.when(s == NS - 1)
    def _():
        pltpu.sync_copy(acc.at[pl.ds(start, RLAST)], out2d.at[pl.ds(start, RLAST)])


def _deg_body(e_hbm, out_hbm, didx, ones_v, sem, acc):
    c = lax.axis_index("c")
    s = lax.axis_index("s")
    w = c * NS + s

    _fill2d(ones_v, CH, 16, 0.0)
    _zero_acc_rows(ones_v.at[pl.ds(0, ZROWS)], acc, s)
    _fill2d(ones_v, CH, 16, 1.0)
    pltpu.sync_copy(e_hbm.at[1, w], didx)
    plsc.subcore_barrier()

    def chunk(j, _):
        pltpu.async_copy(ones_v, acc.at[didx.at[j]], sem, add=True)
        return 0

    lax.fori_loop(0, NCH, chunk, 0)

    def drain(j, _):
        pltpu.make_async_copy(ones_v, acc.at[didx.at[j]], sem).wait()
        return 0

    lax.fori_loop(0, NCH, drain, 0)
    plsc.subcore_barrier()
    _copy_out_rows(acc, out_hbm.at[c], s)


_deg = functools.partial(
    pl.kernel,
    out_type=jax.ShapeDtypeStruct((NC, N, 16), jnp.float32),
    mesh=_MESH,
    scratch_types=[
        pltpu.VMEM((NCH, CH), jnp.int32),
        pltpu.VMEM((CH, 16), jnp.float32),
        pltpu.SemaphoreType.DMA,
        pltpu.VMEM_SHARED((N, 16), jnp.float32),
    ],
    compiler_params=pltpu.CompilerParams(use_tc_tiling_on_sc=False),
)(_deg_body)


NH = 2
HW = D // NH


NBUF = 6
AHEAD = NBUF // 2
MAIN = (NCH // NBUF) * NBUF


def _copy_out_cols(acc, out2d, s, off):
    start = pl.multiple_of(RMAIN * s, RMAIN)

    @pl.when(s < NS - 1)
    def _():
        pltpu.sync_copy(
            acc.at[pl.ds(start, RMAIN)],
            out2d.at[pl.ds(start, RMAIN), pl.ds(off, HW)],
        )

    @pl.when(s == NS - 1)
    def _():
        pltpu.sync_copy(
            acc.at[pl.ds(start, RLAST)],
            out2d.at[pl.ds(start, RLAST), pl.ds(off, HW)],
        )


def _scat_body(
    e_hbm, y_hbm, out_hbm,
    sidx, didx, *rest,
):
    bufs = rest[:NBUF]
    gsems = rest[NBUF : 2 * NBUF]
    ssems = rest[2 * NBUF : 3 * NBUF]
    acc = rest[3 * NBUF]
    c = lax.axis_index("c")
    s = lax.axis_index("s")
    w = c * NS + s
    buf0 = bufs[0]

    pltpu.sync_copy(e_hbm.at[0, w], sidx)
    pltpu.sync_copy(e_hbm.at[1, w], didx)

    for half in range(NH):
        y_h = y_hbm.at[pl.ds(half, 2 * N - 1)]

        _fill2d(buf0, CH, HW, 0.0)
        _zero_acc_rows(buf0.at[pl.ds(0, ZROWS)], acc, s)

        for j0 in range(AHEAD):
            pltpu.async_copy(y_h.at[sidx.at[j0]], bufs[j0 % NBUF], gsems[j0 % NBUF])
        plsc.subcore_barrier()

        def step(jj, _):
            for b in range(NBUF):
                j = jj * NBUF + b
                b2 = (b + AHEAD) % NBUF
                pltpu.make_async_copy(y_h.at[sidx.at[j]], bufs[b], gsems[b]).wait()
                pltpu.async_copy(bufs[b], acc.at[didx.at[j]], ssems[b], add=True)

                @pl.when(j >= AHEAD)
                def _():
                    pltpu.make_async_copy(
                        bufs[b2], acc.at[didx.at[j - AHEAD]], ssems[b2]
                    ).wait()

                @pl.when(j + AHEAD < NCH)
                def _():
                    pltpu.async_copy(y_h.at[sidx.at[j + AHEAD]], bufs[b2], gsems[b2])

            return 0

        lax.fori_loop(0, MAIN // NBUF, step, 0)
        for j in range(MAIN, NCH):
            b = j % NBUF
            pltpu.make_async_copy(y_h.at[sidx.at[j]], bufs[b], gsems[b]).wait()
            pltpu.async_copy(bufs[b], acc.at[didx.at[j]], ssems[b], add=True)
        for j in range(MAIN - AHEAD, NCH):
            b = j % NBUF
            pltpu.make_async_copy(bufs[b], acc.at[didx.at[j]], ssems[b]).wait()
        plsc.subcore_barrier()
        _copy_out_cols(acc, out_hbm.at[c], s, half * HW)


_scat = functools.partial(
    pl.kernel,
    out_type=jax.ShapeDtypeStruct((NC, N, D), jnp.float32),
    mesh=_MESH,
    scratch_types=[
        pltpu.VMEM((NCH, CH), jnp.int32),
        pltpu.VMEM((NCH, CH), jnp.int32),
        *([pltpu.VMEM((CH, HW), jnp.float32)] * NBUF),
        *([pltpu.SemaphoreType.DMA] * (2 * NBUF)),
        pltpu.VMEM_SHARED((N, HW), jnp.float32),
    ],
    compiler_params=pltpu.CompilerParams(use_tc_tiling_on_sc=False),
)(_scat_body)


R = 2000
GRID = N // R


def _dinv_block(degp_ref):
    d = degp_ref[0, :, 0:1] + degp_ref[1, :, 0:1] + 1.0
    return lax.rsqrt(d)


def _tcmm_body(x_ref, w_ref, xw_ref):
    xw_ref[...] = jnp.dot(
        x_ref[...], w_ref[...], preferred_element_type=jnp.float32
    )


_tcmm = pl.pallas_call(
    _tcmm_body,
    grid=(GRID,),
    in_specs=[
        pl.BlockSpec((R, D), lambda i: (i, 0)),
        pl.BlockSpec((D, D), lambda i: (0, 0)),
    ],
    out_specs=pl.BlockSpec((R, D), lambda i: (i, 0)),
    out_shape=jax.ShapeDtypeStruct((N, D), jnp.float32),
)


def _tc1_body(xw_ref, degp_ref, y_ref):
    dinv = _dinv_block(degp_ref)
    y_ref[...] = xw_ref[...] * dinv


_tc1 = pl.pallas_call(
    _tc1_body,
    grid=(GRID,),
    in_specs=[
        pl.BlockSpec((R, D), lambda i: (i, 0)),
        pl.BlockSpec((NC, R, 16), lambda i: (0, i, 0)),
    ],
    out_specs=pl.BlockSpec((R, D), lambda i: (i, 0)),
    out_shape=jax.ShapeDtypeStruct((N, D), jnp.float32),
)


def _tc2_body(y1_ref, acc_ref, degp_ref, b_ref, w_ref, y2_ref):
    dinv = _dinv_block(degp_ref)
    tot = acc_ref[0] + acc_ref[1] + y1_ref[...]
    h = jnp.maximum(tot * dinv + b_ref[...], 0.0)
    y2_ref[...] = (
        jnp.dot(h, w_ref[...], preferred_element_type=jnp.float32) * dinv
    )


_tc2 = pl.pallas_call(
    _tc2_body,
    grid=(GRID,),
    in_specs=[
        pl.BlockSpec((R, D), lambda i: (i, 0)),
        pl.BlockSpec((NC, R, D), lambda i: (0, i, 0)),
        pl.BlockSpec((NC, R, 16), lambda i: (0, i, 0)),
        pl.BlockSpec((1, D), lambda i: (0, 0)),
        pl.BlockSpec((D, D), lambda i: (0, 0)),
    ],
    out_specs=pl.BlockSpec((R, D), lambda i: (i, 0)),
    out_shape=jax.ShapeDtypeStruct((N, D), jnp.float32),
)


def _tc3_body(y2_ref, acc_ref, degp_ref, b_ref, o_ref):
    dinv = _dinv_block(degp_ref)
    tot = acc_ref[0] + acc_ref[1] + y2_ref[...]
    t = tot * dinv + b_ref[...]
    o_ref[...] = jax.nn.sigmoid(t) * 0.8 + 0.1


_tc3 = pl.pallas_call(
    _tc3_body,
    grid=(GRID,),
    in_specs=[
        pl.BlockSpec((R, D), lambda i: (i, 0)),
        pl.BlockSpec((NC, R, D), lambda i: (0, i, 0)),
        pl.BlockSpec((NC, R, 16), lambda i: (0, i, 0)),
        pl.BlockSpec((1, D), lambda i: (0, 0)),
    ],
    out_specs=pl.BlockSpec((R, D), lambda i: (i, 0)),
    out_shape=jax.ShapeDtypeStruct((N, D), jnp.float32),
)


@jax.jit
def kernel(x, edge_index, W1, b1, W2, b2):
    flat = jnp.concatenate([edge_index[0] * 2, edge_index[1]])
    edge3 = flat.reshape(2, NW, NCH, CH)
    b1r = b1.reshape(1, D)
    b2r = b2.reshape(1, D)

    xw1 = _tcmm(x, W1)
    degp = _deg(edge3)
    y1 = _tc1(xw1, degp)
    acc1 = _scat(edge3, y1.reshape(2 * N, HW))
    y2 = _tc2(y1, acc1, degp, b1r, W2)
    acc2 = _scat(edge3, y2.reshape(2 * N, HW))
    return _tc3(y2, acc2, degp, b2r)
